# Initial kernel scaffold; baseline (speedup 1.0000x reference)
import jax
import jax.numpy as jnp
from jax.experimental import pallas as pl


def kernel(feature, edge_index, W, b):
    raise NotImplementedError("write your pallas kernel here")


# speedup vs baseline: 4.6539x; 4.6539x over previous
import functools

import jax
import jax.numpy as jnp
from jax import lax
from jax.experimental import pallas as pl
from jax.experimental.pallas import tpu as pltpu
from jax.experimental.pallas import tpu_sc as plsc

N_NODES = 10000
N_EDGES = 320000
D = 128

NUM_CORES = 2
NUM_SUBCORES = 16
NW = NUM_CORES * NUM_SUBCORES
CHUNK = 128
ROWS_PER_TILE = 79
E_PAD = NW * ROWS_PER_TILE * CHUNK
ACC_ROWS = 10240
ZERO_ROWS = ACC_ROWS // NUM_SUBCORES


def _sc_segment_sum(feature, srcp, dstp, zeros):
    mesh = plsc.VectorSubcoreMesh(core_axis_name="c", subcore_axis_name="s")

    @functools.partial(
        pl.kernel,
        out_type=jax.ShapeDtypeStruct((NUM_CORES, ACC_ROWS, D), jnp.float32),
        mesh=mesh,
        scratch_types=[
            pltpu.VMEM((ROWS_PER_TILE, CHUNK), jnp.int32),
            pltpu.VMEM((ROWS_PER_TILE, CHUNK), jnp.int32),
            pltpu.VMEM((CHUNK, D), jnp.float32),
            pltpu.VMEM_SHARED((ACC_ROWS, D), jnp.float32),
            pltpu.SemaphoreType.DMA,
        ],
    )
    def k(feature_hbm, src_hbm, dst_hbm, zeros_hbm, out_hbm,
          src_v, dst_v, rows_v, acc, sem):
        c = lax.axis_index("c")
        s = lax.axis_index("s")
        w = c * NUM_SUBCORES + s
        pltpu.sync_copy(zeros_hbm, acc.at[pl.ds(s * ZERO_ROWS, ZERO_ROWS)])
        pltpu.sync_copy(src_hbm.at[w], src_v)
        pltpu.sync_copy(dst_hbm.at[w], dst_v)
        plsc.subcore_barrier()

        def body(j, carry):
            pltpu.async_copy(feature_hbm.at[src_v.at[j]], rows_v, sem).wait()
            pltpu.sync_copy(rows_v, acc.at[dst_v.at[j]], add=True)
            return carry

        lax.fori_loop(0, ROWS_PER_TILE, body, 0)
        plsc.subcore_barrier()
        pltpu.sync_copy(acc.at[pl.ds(s * ZERO_ROWS, ZERO_ROWS)],
                        out_hbm.at[c, pl.ds(s * ZERO_ROWS, ZERO_ROWS)])

    return k(feature, srcp, dstp, zeros)


def _tc_linear(partials, W, b):
    blk = 1000
    grid = N_NODES // blk

    def body(p_ref, w_ref, b_ref, o_ref):
        x = p_ref[0] + p_ref[1]
        y = lax.dot_general(x, w_ref[...], (((1,), (1,)), ((), ())),
                            preferred_element_type=jnp.float32,
                            precision=lax.Precision.HIGHEST)
        o_ref[...] = y + b_ref[...]

    return pl.pallas_call(
        body,
        grid=(grid,),
        in_specs=[
            pl.BlockSpec((NUM_CORES, blk, D), lambda i: (0, i, 0)),
            pl.BlockSpec((D, D), lambda i: (0, 0)),
            pl.BlockSpec((1, D), lambda i: (0, 0)),
        ],
        out_specs=pl.BlockSpec((blk, D), lambda i: (i, 0)),
        out_shape=jax.ShapeDtypeStruct((N_NODES, D), jnp.float32),
    )(partials, W, b.reshape(1, D))


def kernel(feature, edge_index, W, b):
    src = edge_index[0].astype(jnp.int32)
    dst = edge_index[1].astype(jnp.int32)
    pad = E_PAD - N_EDGES
    src = jnp.concatenate([src, jnp.zeros((pad,), jnp.int32)])
    dst = jnp.concatenate([dst, jnp.full((pad,), N_NODES, jnp.int32)])
    srcp = src.reshape(NW, ROWS_PER_TILE, CHUNK)
    dstp = dst.reshape(NW, ROWS_PER_TILE, CHUNK)
    zeros = jnp.zeros((ZERO_ROWS, D), jnp.float32)
    partials = _sc_segment_sum(feature, srcp, dstp, zeros)
    return _tc_linear(partials, W, b)


# speedup vs baseline: 4.8077x; 1.0330x over previous
import functools

import jax
import jax.numpy as jnp
from jax import lax
from jax.experimental import pallas as pl
from jax.experimental.pallas import tpu as pltpu
from jax.experimental.pallas import tpu_sc as plsc

N_NODES = 10000
N_EDGES = 320000
D = 128

NUM_CORES = 2
NUM_SUBCORES = 16
NW = NUM_CORES * NUM_SUBCORES
CHUNK = 128
ROWS_PER_TILE = 79
E_PAD = NW * ROWS_PER_TILE * CHUNK
ACC_ROWS = 10240
ZERO_ROWS = ACC_ROWS // NUM_SUBCORES


def _sc_segment_sum(feature, idxp, zeros):
    mesh = plsc.VectorSubcoreMesh(core_axis_name="c", subcore_axis_name="s")

    @functools.partial(
        pl.kernel,
        out_type=jax.ShapeDtypeStruct((NUM_CORES, ACC_ROWS, D), jnp.float32),
        mesh=mesh,
        scratch_types=[
            pltpu.VMEM((2, CHUNK), jnp.int32),
            pltpu.VMEM((2, CHUNK), jnp.int32),
            pltpu.VMEM((CHUNK, D), jnp.float32),
            pltpu.VMEM((CHUNK, D), jnp.float32),
            pltpu.VMEM_SHARED((ACC_ROWS, D), jnp.float32),
            pltpu.SemaphoreType.DMA,
            pltpu.SemaphoreType.DMA,
            pltpu.SemaphoreType.DMA,
        ],
    )
    def k(feature_hbm, idx_hbm, zeros_hbm, out_hbm,
          ibuf0, ibuf1, buf0, buf1, acc, isem, gsem, ssem):
        c = lax.axis_index("c")
        s = lax.axis_index("s")
        w = c * NUM_SUBCORES + s
        pltpu.sync_copy(zeros_hbm, acc.at[pl.ds(s * ZERO_ROWS, ZERO_ROWS)])
        plsc.subcore_barrier()

        def fetch_idx(j, ibuf):
            pltpu.async_copy(idx_hbm.at[w, j], ibuf, isem)

        def wait_idx(ibuf):
            pltpu.make_async_copy(idx_hbm.at[w, 0], ibuf, isem).wait()

        def gather(ibuf, buf):
            pltpu.async_copy(feature_hbm.at[ibuf.at[0]], buf, gsem).wait()

        def scatter(ibuf, buf):
            pltpu.async_copy(buf, acc.at[ibuf.at[1]], ssem, add=True)

        def wait_scatter(ibuf, buf):
            pltpu.make_async_copy(buf, acc.at[ibuf.at[1]], ssem).wait()

        fetch_idx(0, ibuf0)

        def body(i, carry):
            j = 2 * i
            wait_idx(ibuf0)
            gather(ibuf0, buf0)
            scatter(ibuf0, buf0)

            @pl.when(i > 0)
            def _():
                wait_scatter(ibuf1, buf1)

            fetch_idx(j + 1, ibuf1)
            wait_idx(ibuf1)
            gather(ibuf1, buf1)
            scatter(ibuf1, buf1)
            wait_scatter(ibuf0, buf0)
            fetch_idx(jnp.minimum(j + 2, ROWS_PER_TILE - 1), ibuf0)
            return carry

        lax.fori_loop(0, (ROWS_PER_TILE - 1) // 2, body, 0)
        wait_idx(ibuf0)
        gather(ibuf0, buf0)
        scatter(ibuf0, buf0)
        wait_scatter(ibuf1, buf1)
        wait_scatter(ibuf0, buf0)
        plsc.subcore_barrier()
        pltpu.sync_copy(acc.at[pl.ds(s * ZERO_ROWS, ZERO_ROWS)],
                        out_hbm.at[c, pl.ds(s * ZERO_ROWS, ZERO_ROWS)])

    return k(feature, idxp, zeros)


def _tc_linear(partials, W, b):
    blk = 1000
    grid = N_NODES // blk

    def body(p_ref, w_ref, b_ref, o_ref):
        x = p_ref[0] + p_ref[1]
        y = lax.dot_general(x, w_ref[...], (((1,), (1,)), ((), ())),
                            preferred_element_type=jnp.float32,
                            precision=lax.Precision.HIGHEST)
        o_ref[...] = y + b_ref[...]

    return pl.pallas_call(
        body,
        grid=(grid,),
        in_specs=[
            pl.BlockSpec((NUM_CORES, blk, D), lambda i: (0, i, 0)),
            pl.BlockSpec((D, D), lambda i: (0, 0)),
            pl.BlockSpec((1, D), lambda i: (0, 0)),
        ],
        out_specs=pl.BlockSpec((blk, D), lambda i: (i, 0)),
        out_shape=jax.ShapeDtypeStruct((N_NODES, D), jnp.float32),
    )(partials, W, b.reshape(1, D))


def kernel(feature, edge_index, W, b):
    src = edge_index[0].astype(jnp.int32)
    dst = edge_index[1].astype(jnp.int32)
    pad = E_PAD - N_EDGES
    src = jnp.concatenate([src, jnp.zeros((pad,), jnp.int32)])
    dst = jnp.concatenate([dst, jnp.full((pad,), N_NODES, jnp.int32)])
    idxp = jnp.stack([src.reshape(NW, ROWS_PER_TILE, CHUNK),
                      dst.reshape(NW, ROWS_PER_TILE, CHUNK)], axis=2)
    zeros = jnp.zeros((ZERO_ROWS, D), jnp.float32)
    partials = _sc_segment_sum(feature, idxp, zeros)
    return _tc_linear(partials, W, b)
